# baseline (speedup 1.0000x reference)
import functools

import jax
import jax.numpy as jnp
from jax import lax
from jax.experimental import pallas as pl
from jax.experimental.pallas import tpu as pltpu
from jax.experimental.pallas import tpu_sc as plsc

BATCH = 4096
HIST = 200
EMBED_DIM = 64
NDT = EMBED_DIM // 8
NCT = BATCH // 128
XBUF = 4
GBUF = 3
OBUF = 3

_mesh = plsc.VectorSubcoreMesh(
    core_axis_name="c", subcore_axis_name="s", num_cores=2, num_subcores=16
)


@functools.partial(
    pl.kernel,
    out_type=jax.ShapeDtypeStruct((HIST, NDT, NCT, 8, 128), jnp.float32),
    mesh=_mesh,
    scratch_types=[
        pltpu.VMEM((HIST, 128), jnp.int32),
        pltpu.VMEM((XBUF, NDT, 8, 128), jnp.float32),
        pltpu.VMEM((GBUF, 128, EMBED_DIM), jnp.float32),
        pltpu.VMEM((OBUF, NDT, 8, 128), jnp.float32),
        pltpu.SemaphoreType.DMA,
        pltpu.SemaphoreType.DMA,
        pltpu.SemaphoreType.DMA,
    ],
    compiler_params=pltpu.CompilerParams(
        use_tc_tiling_on_sc=False, needs_layout_passes=False
    ),
)
def _embed_add(xv, idxT, table, outv, idx_v, x_v, g_v, o_v, sem_x, sem_g, sem_s):
    w = lax.axis_index("s") * 2 + lax.axis_index("c")

    pltpu.sync_copy(idxT.at[:, pl.ds(w * 128, 128)], idx_v)

    for h in range(3):
        pltpu.async_copy(xv.at[h, :, w], x_v.at[h], sem_x)
    for h in range(2):
        pltpu.async_copy(table.at[idx_v.at[h]], g_v.at[h], sem_g)

    lanes = lax.iota(jnp.int32, 16)
    pats = [(lanes + j) & 15 for j in range(16)]

    def body(h, _):
        xb = lax.rem(h, XBUF)
        gb = lax.rem(h, GBUF)
        ob = lax.rem(h, OBUF)

        @pl.when(h + 3 < HIST)
        def _load_ahead():
            pltpu.async_copy(xv.at[h + 3, :, w], x_v.at[lax.rem(h + 3, XBUF)], sem_x)

        @pl.when(h + 2 < HIST)
        def _gather_ahead():
            pltpu.async_copy(
                table.at[idx_v.at[h + 2]], g_v.at[lax.rem(h + 2, GBUF)], sem_g
            )

        pltpu.make_async_copy(xv.at[h, :, w], x_v.at[xb], sem_x).wait()
        pltpu.make_async_copy(table.at[idx_v.at[h]], g_v.at[gb], sem_g).wait()

        @pl.when(h >= OBUF)
        def _drain_store():
            pltpu.make_async_copy(
                o_v.at[ob], outv.at[h - OBUF, :, w], sem_s
            ).wait()

        def rt_body(rt, _):
            rv = lanes + rt * 16
            for dt2 in range(4):
                for j in range(16):
                    dvec = pats[j] + dt2 * 16
                    dtv = lax.shift_right_logical(dvec, 3)
                    drv = dvec & 7
                    gcol = plsc.load_gather(g_v.at[gb], [rv, dvec])
                    xold = plsc.load_gather(x_v.at[xb], [dtv, drv, rv])
                    plsc.store_scatter(o_v.at[ob], [dtv, drv, rv], xold + gcol)
            return ()

        lax.fori_loop(0, 8, rt_body, ())

        pltpu.async_copy(o_v.at[ob], outv.at[h, :, w], sem_s)
        return ()

    lax.fori_loop(0, HIST, body, ())

    for h in range(HIST - OBUF, HIST):
        pltpu.make_async_copy(
            o_v.at[h % OBUF], outv.at[h, :, w], sem_s
        ).wait()


NUM_EMB = 1000000
TBLK = 1024
NTBLK = -(-NUM_EMB // TBLK)


def _transpose_body(tin_ref, tout_ref):
    tout_ref[...] = tin_ref[...].T


_table_rm = pl.pallas_call(
    _transpose_body,
    grid=(NTBLK,),
    in_specs=[pl.BlockSpec((EMBED_DIM, TBLK), lambda i: (0, i))],
    out_specs=pl.BlockSpec((TBLK, EMBED_DIM), lambda i: (i, 0)),
    out_shape=jax.ShapeDtypeStruct((NUM_EMB, EMBED_DIM), jnp.float32),
)


def kernel(x, in_chan_matrix, embed_weight):
    xv = (
        x.transpose(1, 2, 0)
        .reshape(HIST, NDT, 8, NCT, 128)
        .transpose(0, 1, 3, 2, 4)
    )
    idxT = in_chan_matrix.astype(jnp.int32).T
    table_rm = _table_rm(embed_weight.T)
    outv = _embed_add(xv, idxT, table_rm)
    out = (
        outv.transpose(0, 1, 3, 2, 4)
        .reshape(HIST, EMBED_DIM, BATCH)
        .transpose(2, 0, 1)
    )
    return out


# speedup vs baseline: 1.4399x; 1.4399x over previous
import functools

import jax
import jax.numpy as jnp
from jax import lax
from jax.experimental import pallas as pl
from jax.experimental.pallas import tpu as pltpu
from jax.experimental.pallas import tpu_sc as plsc

BATCH = 4096
HIST = 200
EMBED_DIM = 64
NDT = EMBED_DIM // 8
NCT = BATCH // 128
XBUF = 4
GBUF = 4
OBUF = 3

_mesh = plsc.VectorSubcoreMesh(
    core_axis_name="c", subcore_axis_name="s", num_cores=2, num_subcores=16
)


@functools.partial(
    pl.kernel,
    out_type=jax.ShapeDtypeStruct((HIST, NDT, NCT, 8, 128), jnp.float32),
    mesh=_mesh,
    scratch_types=[
        pltpu.VMEM((HIST, 128), jnp.int32),
        pltpu.VMEM((XBUF, NDT, 8, 128), jnp.float32),
        pltpu.VMEM((GBUF, 128, EMBED_DIM), jnp.float32),
        pltpu.VMEM((OBUF, NDT, 8, 128), jnp.float32),
        pltpu.SemaphoreType.DMA,
        pltpu.SemaphoreType.DMA,
        pltpu.SemaphoreType.DMA,
    ],
    compiler_params=pltpu.CompilerParams(
        use_tc_tiling_on_sc=False, needs_layout_passes=False
    ),
)
def _embed_add(xv, idxT, table, outv, idx_v, x_v, g_v, o_v, sem_x, sem_g, sem_s):
    w = lax.axis_index("s") * 2 + lax.axis_index("c")

    pltpu.sync_copy(idxT.at[:, pl.ds(w * 128, 128)], idx_v)

    for h in range(3):
        pltpu.async_copy(xv.at[h, :, w], x_v.at[h], sem_x)
    for h in range(3):
        pltpu.async_copy(table.at[idx_v.at[h]], g_v.at[h], sem_g)

    lanes = lax.iota(jnp.int32, 16)
    pats = [(lanes + j) & 15 for j in range(16)]

    def body(h, _):
        xb = lax.rem(h, XBUF)
        gb = lax.rem(h, GBUF)
        ob = lax.rem(h, OBUF)

        @pl.when(h + 3 < HIST)
        def _load_ahead():
            pltpu.async_copy(xv.at[h + 3, :, w], x_v.at[lax.rem(h + 3, XBUF)], sem_x)

        @pl.when(h + 3 < HIST)
        def _gather_ahead():
            pltpu.async_copy(
                table.at[idx_v.at[h + 3]], g_v.at[lax.rem(h + 3, GBUF)], sem_g
            )

        pltpu.make_async_copy(xv.at[h, :, w], x_v.at[xb], sem_x).wait()
        pltpu.make_async_copy(table.at[idx_v.at[h]], g_v.at[gb], sem_g).wait()

        @pl.when(h >= OBUF)
        def _drain_store():
            pltpu.make_async_copy(
                o_v.at[ob], outv.at[h - OBUF, :, w], sem_s
            ).wait()

        def rt_body(rt, _):
            rv = lanes + rt * 16
            for dt2 in range(4):
                for j in range(16):
                    dvec = pats[j] + dt2 * 16
                    dtv = lax.shift_right_logical(dvec, 3)
                    drv = dvec & 7
                    gcol = plsc.load_gather(g_v.at[gb], [rv, dvec])
                    xold = plsc.load_gather(x_v.at[xb], [dtv, drv, rv])
                    plsc.store_scatter(o_v.at[ob], [dtv, drv, rv], xold + gcol)
            return ()

        lax.fori_loop(0, 8, rt_body, ())

        pltpu.async_copy(o_v.at[ob], outv.at[h, :, w], sem_s)
        return ()

    lax.fori_loop(0, HIST, body, ())

    for h in range(HIST - OBUF, HIST):
        pltpu.make_async_copy(
            o_v.at[h % OBUF], outv.at[h, :, w], sem_s
        ).wait()


def kernel(x, in_chan_matrix, embed_weight):
    xv = (
        x.transpose(1, 2, 0)
        .reshape(HIST, NDT, 8, NCT, 128)
        .transpose(0, 1, 3, 2, 4)
    )
    idxT = in_chan_matrix.astype(jnp.int32).T
    outv = _embed_add(xv, idxT, embed_weight)
    out = (
        outv.transpose(0, 1, 3, 2, 4)
        .reshape(HIST, EMBED_DIM, BATCH)
        .transpose(2, 0, 1)
    )
    return out
